# baseline (speedup 1.0000x reference)
import functools

import jax
import jax.numpy as jnp
from jax import lax
from jax.experimental import pallas as pl
from jax.experimental.pallas import tpu as pltpu
from jax.experimental.pallas import tpu_sc as plsc

C = 1000
B = 16384
NC, NS, L = 2, 16, 16
NW = NC * NS
ROWS_PER_W = B // NW
CHUNK = 64
NCHUNK = ROWS_PER_W // CHUNK
ZBUF = CHUNK * C
IDX_W = 128
NIDX = ROWS_PER_W // IDX_W

_mesh = plsc.VectorSubcoreMesh(core_axis_name="c", subcore_axis_name="s")


@functools.partial(
    pl.kernel,
    mesh=_mesh,
    out_type=jax.ShapeDtypeStruct((B * C,), jnp.float32),
    scratch_types=[
        pltpu.VMEM((ROWS_PER_W,), jnp.int32),
        pltpu.VMEM((ZBUF,), jnp.float32),
        pltpu.VMEM((NIDX, IDX_W), jnp.int32),
        pltpu.VMEM((NIDX, IDX_W), jnp.float32),
        pltpu.SemaphoreType.DMA,
        pltpu.SemaphoreType.DMA,
    ],
)
def _onehot_sc(y_hbm, out_hbm, idx_v, zbuf, pos_v, ones_v, zsem, ssem):
    wid = lax.axis_index("s") * NC + lax.axis_index("c")
    base = wid * ROWS_PER_W
    pltpu.sync_copy(y_hbm.at[pl.ds(base, ROWS_PER_W)], idx_v)

    zeros16 = jnp.zeros((L,), jnp.float32)
    ones16 = jnp.ones((L,), jnp.float32)

    def zero_body(i, carry):
        for u in range(4):
            zbuf[pl.ds((i * 4 + u) * L, L)] = zeros16
        return carry

    lax.fori_loop(0, ZBUF // (4 * L), zero_body, 0)

    zcopies = []
    for c in range(NCHUNK):
        dst = out_hbm.at[pl.ds((base + c * CHUNK) * C, ZBUF)]
        zcopies.append(pltpu.async_copy(zbuf, dst, zsem))

    iota = lax.iota(jnp.int32, L)
    for j in range(NIDX):
        for k in range(IDX_W // L):
            r = j * IDX_W + k * L
            y16 = idx_v[pl.ds(r, L)]
            pos_v[j, pl.ds(k * L, L)] = (base + r + iota) * C + y16
            ones_v[j, pl.ds(k * L, L)] = ones16

    for cp in zcopies:
        cp.wait()

    scopies = []
    for j in range(NIDX):
        scopies.append(
            pltpu.async_copy(ones_v.at[j], out_hbm.at[pos_v.at[j]], ssem)
        )
    for cp in scopies:
        cp.wait()


def kernel(y_n):
    flat = _onehot_sc(y_n)
    return flat.reshape(B, C)
